# baseline (speedup 1.0000x reference)
import jax
import jax.numpy as jnp
from jax import lax
from jax.experimental import pallas as pl
from jax.experimental.pallas import tpu as pltpu
from jax.experimental.pallas import tpu_sc as plsc

TT = 131072
NT = 32768
D = 256
OD = 2 * D
NC = 2
NS = 16
NW = NC * NS
CTOK = 256
PCHUNK = NT // NW // CTOK
NCUT = NT // CTOK + 1
B = 96
L = 16
RB = 4096


def _sc_body(trace_hbm, tid_hbm, w_hbm, bounds_hbm, out_hbm,
             tbufs, idbufs, wbufs, ixbuf, bounds_v, acc, sems):
    c = lax.axis_index("c")
    s = lax.axis_index("s")
    wid = s * NC + c
    lanes = lax.iota(jnp.int32, L)
    zero16 = jnp.zeros((L,), jnp.float32)

    pltpu.sync_copy(bounds_hbm.at[pl.ds(wid * L, L)], bounds_v)
    bvec = bounds_v[...]

    for p in range(PCHUNK):
        base_tok = (wid * PCHUNK + p) * CTOK
        lo = bvec[p]
        hi = bvec[p + 1]

        def zrow(r, carry):
            rsplat = jnp.zeros((L,), jnp.int32) + r
            for j in range(D // L):
                plsc.store_scatter(acc, [rsplat, j * L + lanes], zero16)
            return carry

        lax.fori_loop(0, CTOK, zrow, 0)

        estart = lo // 8 * 8
        nblk = (jnp.maximum(hi - estart, 0) + B - 1) // B
        npair = (nblk + 1) // 2

        run0 = (jnp.zeros((L,), jnp.int32),) + tuple(
            jnp.zeros((L,), jnp.float32) for _ in range(D // L))

        def eblk(b):
            return jnp.minimum(estart + b * B, TT - B)

        def start(buf, b):
            pltpu.async_copy(trace_hbm.at[pl.ds(eblk(b) * D, B * D)],
                             tbufs[buf], sems[buf])
            pltpu.async_copy(tid_hbm.at[pl.ds(eblk(b), B)], idbufs[buf],
                             sems[buf])
            pltpu.async_copy(w_hbm.at[pl.ds(eblk(b), B)], wbufs[buf],
                             sems[buf])

        def drain(buf, b):
            pltpu.make_async_copy(trace_hbm.at[pl.ds(eblk(b) * D, B * D)],
                                  tbufs[buf], sems[buf]).wait()
            pltpu.make_async_copy(tid_hbm.at[pl.ds(eblk(b), B)], idbufs[buf],
                                  sems[buf]).wait()
            pltpu.make_async_copy(w_hbm.at[pl.ds(eblk(b), B)], wbufs[buf],
                                  sems[buf]).wait()

        def process(buf, b, carry):
            tbuf, idbuf, wbuf = tbufs[buf], idbufs[buf], wbufs[buf]
            nominal = estart + b * B
            e = eblk(b)
            wlo = jnp.maximum(lo, nominal)
            whi = jnp.minimum(hi, nominal + B)
            for grp in range(B // L):
                g = e + grp * L + lanes
                tid16 = idbuf[pl.ds(grp * L, L)]
                w16 = wbuf[pl.ds(grp * L, L)]
                valid = (g >= wlo) & (g < whi)
                wbuf[pl.ds(grp * L, L)] = jnp.where(valid, w16, 0.0)
                ixbuf[pl.ds(grp * L, L)] = jnp.clip(tid16 - base_tok, 0,
                                                    CTOK - 1)

            def ev(i, run):
                prev, accv = run[0], run[1:]
                isplat = jnp.zeros((L,), jnp.int32) + i
                wsp = plsc.load_gather(wbuf, [isplat])
                ltok = plsc.load_gather(ixbuf, [isplat])
                changed = ltok != prev
                ibase = i * D
                out = [ltok]
                for j in range(D // L):
                    colv = j * L + lanes
                    plsc.addupdate_scatter(acc, [prev, colv], accv[j],
                                           mask=changed)
                    t = tbuf[pl.ds(ibase + j * L, L)]
                    out.append(jnp.where(changed, 0.0, accv[j]) + wsp * t)
                return tuple(out)

            return lax.fori_loop(0, B, ev, carry)

        @pl.when(npair > 0)
        def _prime():
            start(0, 0)
            start(1, 1)

        def pair(k, carry):
            b0 = 2 * k
            drain(0, b0)
            carry = process(0, b0, carry)

            @pl.when(k + 1 < npair)
            def _s0():
                start(0, b0 + 2)

            drain(1, b0 + 1)
            carry = process(1, b0 + 1, carry)

            @pl.when(k + 1 < npair)
            def _s1():
                start(1, b0 + 3)

            return carry

        run = lax.fori_loop(0, npair, pair, run0)

        prev, accv = run[0], run[1:]
        for j in range(D // L):
            plsc.addupdate_scatter(acc, [prev, j * L + lanes], accv[j])

        pltpu.sync_copy(acc,
                        out_hbm.at[pl.ds(base_tok, CTOK), pl.ds(D, D)])


def _fill_body(outin_hbm, inp_ref, out_ref):
    del outin_hbm
    out_ref[...] = inp_ref[...]


def kernel(inp_embed, trace_embed, token_ids, weights):
    tid = token_ids.astype(jnp.int32)
    cuts = jnp.arange(0, NT + 1, CTOK, dtype=jnp.int32)
    bounds = jnp.searchsorted(tid, cuts, side="left",
                              method="compare_all").astype(jnp.int32)
    gidx = jnp.minimum(jnp.arange(NW)[:, None] * PCHUNK +
                       jnp.arange(L)[None, :], NCUT - 1)
    bounds = bounds[gidx].reshape(-1)
    mesh = plsc.VectorSubcoreMesh(core_axis_name="c", subcore_axis_name="s",
                                  num_cores=NC, num_subcores=NS)
    half = pl.kernel(
        _sc_body,
        out_type=jax.ShapeDtypeStruct((NT, OD), jnp.float32),
        mesh=mesh,
        compiler_params=pltpu.CompilerParams(needs_layout_passes=False),
        scratch_types=[
            [pltpu.VMEM((B * D,), jnp.float32)] * 2,
            [pltpu.VMEM((B,), jnp.int32)] * 2,
            [pltpu.VMEM((B,), jnp.float32)] * 2,
            pltpu.VMEM((B,), jnp.int32),
            pltpu.VMEM((L,), jnp.int32),
            pltpu.VMEM((CTOK, D), jnp.float32),
            [pltpu.SemaphoreType.DMA] * 2,
        ],
    )(trace_embed.reshape(-1), tid, weights, bounds)
    return pl.pallas_call(
        _fill_body,
        grid=(NT // RB,),
        in_specs=[pl.BlockSpec(memory_space=pl.ANY),
                  pl.BlockSpec((RB, D), lambda i: (i, 0))],
        out_specs=pl.BlockSpec((RB, D), lambda i: (i, 0)),
        out_shape=jax.ShapeDtypeStruct((NT, OD), jnp.float32),
        input_output_aliases={0: 0},
    )(half, inp_embed)


# speedup vs baseline: 1.4179x; 1.4179x over previous
import jax
import jax.numpy as jnp
from jax import lax
from jax.experimental import pallas as pl
from jax.experimental.pallas import tpu as pltpu
from jax.experimental.pallas import tpu_sc as plsc

TT = 131072
NT = 32768
D = 256
OD = 2 * D
NC = 2
NS = 16
NW = NC * NS
CTOK = 256
PCHUNK = NT // NW // CTOK
NCUT = NT // CTOK + 1
B = 96
L = 16
RB = 4096


def _sc_body(trace_hbm, tid_hbm, w_hbm, bounds_hbm, out_hbm,
             tbufs, idbufs, wbufs, ixbuf, bounds_v, acc, sems):
    c = lax.axis_index("c")
    s = lax.axis_index("s")
    wid = s * NC + c
    lanes = lax.iota(jnp.int32, L)
    zero16 = jnp.zeros((L,), jnp.float32)

    pltpu.sync_copy(bounds_hbm.at[pl.ds(wid * L, L)], bounds_v)
    bvec = bounds_v[...]

    for p in range(PCHUNK):
        base_tok = (wid * PCHUNK + p) * CTOK
        lo = bvec[p]
        hi = bvec[p + 1]

        def zrow(r, carry):
            rsplat = jnp.zeros((L,), jnp.int32) + r
            for j in range(D // L):
                plsc.store_scatter(acc, [rsplat, j * L + lanes], zero16)
            return carry

        lax.fori_loop(0, CTOK, zrow, 0)

        estart = lo // 8 * 8
        nblk = (jnp.maximum(hi - estart, 0) + B - 1) // B
        npair = (nblk + 1) // 2

        run0 = (jnp.zeros((L,), jnp.int32),) + tuple(
            jnp.zeros((L,), jnp.float32) for _ in range(D // L))

        def eblk(b):
            return jnp.minimum(estart + b * B, TT - B)

        def start(buf, b):
            pltpu.async_copy(trace_hbm.at[pl.ds(eblk(b), B)], tbufs[buf],
                             sems[buf])
            pltpu.async_copy(tid_hbm.at[pl.ds(eblk(b), B)], idbufs[buf],
                             sems[buf])
            pltpu.async_copy(w_hbm.at[pl.ds(eblk(b), B)], wbufs[buf],
                             sems[buf])

        def drain(buf, b):
            pltpu.make_async_copy(trace_hbm.at[pl.ds(eblk(b), B)], tbufs[buf],
                                  sems[buf]).wait()
            pltpu.make_async_copy(tid_hbm.at[pl.ds(eblk(b), B)], idbufs[buf],
                                  sems[buf]).wait()
            pltpu.make_async_copy(w_hbm.at[pl.ds(eblk(b), B)], wbufs[buf],
                                  sems[buf]).wait()

        def process(buf, b, carry):
            tbuf, idbuf, wbuf = tbufs[buf], idbufs[buf], wbufs[buf]
            nominal = estart + b * B
            e = eblk(b)
            wlo = jnp.maximum(lo, nominal)
            whi = jnp.minimum(hi, nominal + B)
            for grp in range(B // L):
                g = e + grp * L + lanes
                tid16 = idbuf[pl.ds(grp * L, L)]
                w16 = wbuf[pl.ds(grp * L, L)]
                valid = (g >= wlo) & (g < whi)
                wbuf[pl.ds(grp * L, L)] = jnp.where(valid, w16, 0.0)
                ixbuf[pl.ds(grp * L, L)] = jnp.clip(tid16 - base_tok, 0,
                                                    CTOK - 1)

            def ev(i, run):
                prev, accv = run[0], run[1:]
                isplat = jnp.zeros((L,), jnp.int32) + i
                wsp = plsc.load_gather(wbuf, [isplat])
                ltok = plsc.load_gather(ixbuf, [isplat])
                changed = ltok != prev
                out = [ltok]
                for j in range(D // L):
                    colv = j * L + lanes
                    plsc.addupdate_scatter(acc, [prev, colv], accv[j],
                                           mask=changed)
                    t = plsc.load_gather(tbuf, [isplat, colv])
                    out.append(jnp.where(changed, 0.0, accv[j]) + wsp * t)
                return tuple(out)

            return lax.fori_loop(0, B, ev, carry)

        @pl.when(npair > 0)
        def _prime():
            start(0, 0)
            start(1, 1)

        def pair(k, carry):
            b0 = 2 * k
            drain(0, b0)
            carry = process(0, b0, carry)

            @pl.when(k + 1 < npair)
            def _s0():
                start(0, b0 + 2)

            drain(1, b0 + 1)
            carry = process(1, b0 + 1, carry)

            @pl.when(k + 1 < npair)
            def _s1():
                start(1, b0 + 3)

            return carry

        run = lax.fori_loop(0, npair, pair, run0)

        prev, accv = run[0], run[1:]
        for j in range(D // L):
            plsc.addupdate_scatter(acc, [prev, j * L + lanes], accv[j])

        pltpu.sync_copy(acc,
                        out_hbm.at[pl.ds(base_tok, CTOK), pl.ds(D, D)])


def _fill_body(outin_hbm, inp_ref, out_ref):
    del outin_hbm
    out_ref[...] = inp_ref[...]


def kernel(inp_embed, trace_embed, token_ids, weights):
    tid = token_ids.astype(jnp.int32)
    cuts = jnp.arange(0, NT + 1, CTOK, dtype=jnp.int32)
    bounds = jnp.searchsorted(tid, cuts, side="left",
                              method="compare_all").astype(jnp.int32)
    gidx = jnp.minimum(jnp.arange(NW)[:, None] * PCHUNK +
                       jnp.arange(L)[None, :], NCUT - 1)
    bounds = bounds[gidx].reshape(-1)
    mesh = plsc.VectorSubcoreMesh(core_axis_name="c", subcore_axis_name="s",
                                  num_cores=NC, num_subcores=NS)
    half = pl.kernel(
        _sc_body,
        out_type=jax.ShapeDtypeStruct((NT, OD), jnp.float32),
        mesh=mesh,
        compiler_params=pltpu.CompilerParams(needs_layout_passes=False),
        scratch_types=[
            [pltpu.VMEM((B, D), jnp.float32)] * 2,
            [pltpu.VMEM((B,), jnp.int32)] * 2,
            [pltpu.VMEM((B,), jnp.float32)] * 2,
            pltpu.VMEM((B,), jnp.int32),
            pltpu.VMEM((L,), jnp.int32),
            pltpu.VMEM((CTOK, D), jnp.float32),
            [pltpu.SemaphoreType.DMA] * 2,
        ],
    )(trace_embed, tid, weights, bounds)
    return pl.pallas_call(
        _fill_body,
        grid=(NT // RB,),
        in_specs=[pl.BlockSpec(memory_space=pl.ANY),
                  pl.BlockSpec((RB, D), lambda i: (i, 0))],
        out_specs=pl.BlockSpec((RB, D), lambda i: (i, 0)),
        out_shape=jax.ShapeDtypeStruct((NT, OD), jnp.float32),
        input_output_aliases={0: 0},
    )(half, inp_embed)
